# baseline (speedup 1.0000x reference)
import functools

import jax
import jax.numpy as jnp
from jax import lax
from jax.experimental import pallas as pl
from jax.experimental.pallas import tpu as pltpu
from jax.experimental.pallas import tpu_sc as plsc

N = 10000
E = 320000
IN_DIM = 128
H1 = 8
F1 = 16
HID = H1 * F1
NCLS = 64

NC = 2
NS = 16
NW = NC * NS
EPW = E // NW
CH = 40
NCHUNK = EPW // CH
NPAD = 8
RSTRIDE = 624
RSIZE = 640

_f32 = jnp.float32


def _proj1_body(x_ref, w_ref, s_ref, d_ref, hap_ref, adp_ref):
    h = jnp.dot(x_ref[...], w_ref[...], preferred_element_type=_f32)
    asp = jnp.dot(h, s_ref[...], preferred_element_type=_f32)
    hap_ref[...] = jnp.concatenate([h, asp], axis=1)
    adp_ref[...] = jnp.dot(h, d_ref[...], preferred_element_type=_f32)


def _mid_body(ad_ref, w_ref, s_ref, d_ref, e16_ref, hap_ref, adp_ref):
    a = ad_ref[0, :, :HID] + ad_ref[1, :, :HID]
    d = ad_ref[0, :, HID:] + ad_ref[1, :, HID:]
    r = 1.0 / (d + 1e-16)
    out1 = a * jnp.dot(r, e16_ref[...], preferred_element_type=_f32)
    h1e = jnp.where(out1 > 0, out1, jnp.exp(out1) - 1.0)
    h2 = jnp.dot(h1e, w_ref[...], preferred_element_type=_f32)
    asp = jnp.dot(h2, s_ref[...], preferred_element_type=_f32)
    hap_ref[...] = jnp.concatenate([h2, jnp.zeros_like(h2), asp], axis=1)
    adp_ref[...] = jnp.dot(h2, d_ref[...], preferred_element_type=_f32)


def _final_body(ad_ref, e2_ref, out_ref):
    a = ad_ref[0, :, :NCLS] + ad_ref[1, :, :NCLS]
    d = ad_ref[0, :, HID:] + ad_ref[1, :, HID:]
    r = 1.0 / (d + 1e-16)
    z = a * jnp.dot(r, e2_ref[...], preferred_element_type=_f32)
    m = jnp.max(z, axis=1, keepdims=True)
    ez = jnp.exp(z - m)
    out_ref[...] = (z - m) - jnp.log(jnp.sum(ez, axis=1, keepdims=True))


def _edge_pass_body(eidx_hbm, hap_hbm, adp_hbm, zf_hbm, ad_out,
                    idxr, gd2, hbuf2, msgb2,
                    esemA, esemB, gsemA, gsemB, ssemA, ssemB, ad_sh):
    cid = lax.axis_index("c")
    sid = lax.axis_index("s")
    wid = sid * NC + cid
    r0 = sid * RSTRIDE
    esem = (esemA, esemB)
    gsem = (gsemA, gsemB)
    ssem = (ssemA, ssemB)
    crow0 = wid * NCHUNK

    pltpu.sync_copy(zf_hbm.at[pl.ds(r0, RSIZE)], ad_sh.at[pl.ds(r0, RSIZE)])
    plsc.subcore_barrier()

    def issue_idx(cc, par):
        pltpu.async_copy(eidx_hbm.at[crow0 + cc], idxr.at[cc % 4], esem[par])

    def wait_idx(par):
        pltpu.make_async_copy(eidx_hbm.at[0], idxr.at[0], esem[par]).wait()

    def issue_gathers(cc, b):
        s = idxr.at[cc % 4, 0]
        d = idxr.at[cc % 4, 1]
        pltpu.async_copy(hap_hbm.at[s], hbuf2[b], gsem[b])
        pltpu.async_copy(adp_hbm.at[d], gd2[b], gsem[b])

    def wait_gathers(b):
        pltpu.make_async_copy(hap_hbm.at[pl.ds(0, CH)], hbuf2[b],
                              gsem[b]).wait()
        pltpu.make_async_copy(adp_hbm.at[pl.ds(0, CH)], gd2[b],
                              gsem[b]).wait()

    def compute_scatter(cc, b):
        gd, hbuf, msgb = gd2[b], hbuf2[b], msgb2[b]

        @plsc.parallel_loop(0, CH, 1, unroll=4)
        def erow(e):
            a = hbuf[e, pl.ds(HID, 16)] + gd[e]
            a = jnp.where(a > 0, a, 0.2 * a)
            ev = jnp.exp(a)
            msgb[e, pl.ds(HID, 16)] = ev
            for j in range(HID // 16):
                msgb[e, pl.ds(16 * j, 16)] = hbuf[e, pl.ds(16 * j, 16)] * ev[j]
        pltpu.async_copy(msgb, ad_sh.at[idxr.at[cc % 4, 1]], ssem[b],
                         add=True)

    def wait_scatters(b):
        pltpu.make_async_copy(zf_hbm.at[pl.ds(0, CH)], msgb2[b],
                              ssem[b]).wait()

    issue_idx(0, 0)
    issue_idx(1, 1)
    wait_idx(0)
    issue_gathers(0, 0)
    issue_idx(2, 0)
    wait_gathers(0)
    wait_idx(1)
    issue_gathers(1, 1)
    issue_idx(3, 1)
    compute_scatter(0, 0)
    wait_gathers(1)
    wait_idx(0)
    issue_gathers(2, 0)
    compute_scatter(1, 1)

    def body(g, carry):
        c0 = 2 * g
        wait_gathers(0)
        wait_idx(1)
        issue_gathers(c0 + 1, 1)
        wait_scatters(0)
        issue_idx(c0 + 2, 0)
        compute_scatter(c0, 0)
        wait_gathers(1)
        wait_idx(0)
        issue_gathers(c0 + 2, 0)
        wait_scatters(1)
        issue_idx(c0 + 3, 1)
        compute_scatter(c0 + 1, 1)
        return carry
    lax.fori_loop(1, NCHUNK // 2, body, 0)

    wait_gathers(0)
    wait_idx(1)
    wait_scatters(0)
    wait_scatters(1)

    plsc.subcore_barrier()
    pltpu.sync_copy(ad_sh.at[pl.ds(r0, RSIZE)],
                    ad_out.at[cid, pl.ds(r0, RSIZE)])


WF = HID + 16


@functools.lru_cache(maxsize=None)
def _make_edge_pass():
    mesh = plsc.VectorSubcoreMesh(
        core_axis_name="c", subcore_axis_name="s",
        num_cores=NC, num_subcores=NS)
    return pl.kernel(
        _edge_pass_body,
        out_type=jax.ShapeDtypeStruct((NC, N, WF), _f32),
        mesh=mesh,
        compiler_params=pltpu.CompilerParams(use_tc_tiling_on_sc=False),
        scratch_types=[
            pltpu.VMEM((4, 2, CH), jnp.int32),
            (pltpu.VMEM((CH, 16), _f32), pltpu.VMEM((CH, 16), _f32)),
            (pltpu.VMEM((CH, WF), _f32), pltpu.VMEM((CH, WF), _f32)),
            (pltpu.VMEM((CH, WF), _f32), pltpu.VMEM((CH, WF), _f32)),
            pltpu.SemaphoreType.DMA,
            pltpu.SemaphoreType.DMA,
            pltpu.SemaphoreType.DMA,
            pltpu.SemaphoreType.DMA,
            pltpu.SemaphoreType.DMA,
            pltpu.SemaphoreType.DMA,
            pltpu.VMEM_SHARED((N, WF), _f32),
        ],
    )


def kernel(x, edge_index, W1, att_src1, att_dst1, W2, att_src2, att_dst2):
    ei = edge_index.astype(jnp.int32)
    eidx4 = ei.reshape(2, NW, NCHUNK, CH).transpose(1, 2, 0, 3)
    eidx4 = eidx4.reshape(NW * NCHUNK, 2, CH)
    eidx4 = jnp.concatenate(
        [eidx4, jnp.zeros((NPAD, 2, CH), jnp.int32)], axis=0)

    ar = jnp.arange(HID)
    S1 = jnp.zeros((HID, 16), _f32).at[ar, ar // F1].set(att_src1.reshape(-1))
    D1 = jnp.zeros((HID, 16), _f32).at[ar, ar // F1].set(att_dst1.reshape(-1))
    S2 = jnp.broadcast_to(att_src2.reshape(NCLS, 1), (NCLS, 16)).astype(_f32)
    D2 = jnp.broadcast_to(att_dst2.reshape(NCLS, 1), (NCLS, 16)).astype(_f32)
    E16 = jnp.zeros((16, HID), _f32).at[ar // F1, ar].set(1.0)
    E2 = jnp.zeros((16, NCLS), _f32).at[0, :].set(1.0)
    zwf = jnp.zeros((N, WF), _f32)

    R = 1000
    grid = (N // R,)

    hap1, adp1 = pl.pallas_call(
        _proj1_body,
        grid=grid,
        in_specs=[
            pl.BlockSpec((R, IN_DIM), lambda i: (i, 0)),
            pl.BlockSpec((IN_DIM, HID), lambda i: (0, 0)),
            pl.BlockSpec((HID, 16), lambda i: (0, 0)),
            pl.BlockSpec((HID, 16), lambda i: (0, 0)),
        ],
        out_specs=[
            pl.BlockSpec((R, WF), lambda i: (i, 0)),
            pl.BlockSpec((R, 16), lambda i: (i, 0)),
        ],
        out_shape=[
            jax.ShapeDtypeStruct((N, WF), _f32),
            jax.ShapeDtypeStruct((N, 16), _f32),
        ],
    )(x, W1, S1, D1)

    ad1 = _make_edge_pass()(eidx4, hap1, adp1, zwf)

    hap2, adp2 = pl.pallas_call(
        _mid_body,
        grid=grid,
        in_specs=[
            pl.BlockSpec((NC, R, WF), lambda i: (0, i, 0)),
            pl.BlockSpec((HID, NCLS), lambda i: (0, 0)),
            pl.BlockSpec((NCLS, 16), lambda i: (0, 0)),
            pl.BlockSpec((NCLS, 16), lambda i: (0, 0)),
            pl.BlockSpec((16, HID), lambda i: (0, 0)),
        ],
        out_specs=[
            pl.BlockSpec((R, WF), lambda i: (i, 0)),
            pl.BlockSpec((R, 16), lambda i: (i, 0)),
        ],
        out_shape=[
            jax.ShapeDtypeStruct((N, WF), _f32),
            jax.ShapeDtypeStruct((N, 16), _f32),
        ],
    )(ad1, W2, S2, D2, E16)

    ad2 = _make_edge_pass()(eidx4, hap2, adp2, zwf)

    out = pl.pallas_call(
        _final_body,
        grid=grid,
        in_specs=[
            pl.BlockSpec((NC, R, WF), lambda i: (0, i, 0)),
            pl.BlockSpec((16, NCLS), lambda i: (0, 0)),
        ],
        out_specs=pl.BlockSpec((R, NCLS), lambda i: (i, 0)),
        out_shape=jax.ShapeDtypeStruct((N, NCLS), _f32),
    )(ad2, E2)

    return out


# speedup vs baseline: 1.3593x; 1.3593x over previous
import functools

import jax
import jax.numpy as jnp
from jax import lax
from jax.experimental import pallas as pl
from jax.experimental.pallas import tpu as pltpu
from jax.experimental.pallas import tpu_sc as plsc

N = 10000
E = 320000
IN_DIM = 128
H1 = 8
F1 = 16
HID = H1 * F1
NCLS = 64

NC = 2
NS = 16
NW = NC * NS
EPW = E // NW
CH = 80
NCHUNK = EPW // CH
NPAD = 8
RSTRIDE = 624
RSIZE = 640

_f32 = jnp.float32


def _proj1_body(x_ref, w_ref, s_ref, d_ref, hap_ref, adp_ref):
    h = jnp.dot(x_ref[...], w_ref[...], preferred_element_type=_f32)
    asp = jnp.dot(h, s_ref[...], preferred_element_type=_f32)
    hap_ref[...] = jnp.concatenate([h, asp], axis=1)
    adp_ref[...] = jnp.dot(h, d_ref[...], preferred_element_type=_f32)


def _mid_body(ad_ref, w_ref, s_ref, d_ref, e16_ref, hap_ref, adp_ref):
    a = ad_ref[0, :, :HID] + ad_ref[1, :, :HID]
    d = ad_ref[0, :, HID:] + ad_ref[1, :, HID:]
    r = 1.0 / (d + 1e-16)
    out1 = a * jnp.dot(r, e16_ref[...], preferred_element_type=_f32)
    h1e = jnp.where(out1 > 0, out1, jnp.exp(out1) - 1.0)
    h2 = jnp.dot(h1e, w_ref[...], preferred_element_type=_f32)
    asp = jnp.dot(h2, s_ref[...], preferred_element_type=_f32)
    hap_ref[...] = jnp.concatenate([h2, jnp.zeros_like(h2), asp], axis=1)
    adp_ref[...] = jnp.dot(h2, d_ref[...], preferred_element_type=_f32)


def _final_body(ad_ref, e2_ref, out_ref):
    a = ad_ref[0, :, :NCLS] + ad_ref[1, :, :NCLS]
    d = ad_ref[0, :, HID:] + ad_ref[1, :, HID:]
    r = 1.0 / (d + 1e-16)
    z = a * jnp.dot(r, e2_ref[...], preferred_element_type=_f32)
    m = jnp.max(z, axis=1, keepdims=True)
    ez = jnp.exp(z - m)
    out_ref[...] = (z - m) - jnp.log(jnp.sum(ez, axis=1, keepdims=True))


def _edge_pass_body(eidx_hbm, hap_hbm, adp_hbm, zf_hbm, ad_out,
                    idxr, gd2, hbuf2,
                    esemA, esemB, gsemA, gsemB, ssemA, ssemB, ad_sh):
    cid = lax.axis_index("c")
    sid = lax.axis_index("s")
    wid = sid * NC + cid
    r0 = sid * RSTRIDE
    esem = (esemA, esemB)
    gsem = (gsemA, gsemB)
    ssem = (ssemA, ssemB)
    crow0 = wid * NCHUNK

    pltpu.sync_copy(zf_hbm.at[pl.ds(r0, RSIZE)], ad_sh.at[pl.ds(r0, RSIZE)])
    plsc.subcore_barrier()

    def issue_idx(cc, par):
        pltpu.async_copy(eidx_hbm.at[crow0 + cc], idxr.at[cc % 4], esem[par])

    def wait_idx(par):
        pltpu.make_async_copy(eidx_hbm.at[0], idxr.at[0], esem[par]).wait()

    def issue_gathers(cc, b):
        s = idxr.at[cc % 4, 0]
        d = idxr.at[cc % 4, 1]
        pltpu.async_copy(hap_hbm.at[s], hbuf2[b], gsem[b])
        pltpu.async_copy(adp_hbm.at[d], gd2[b], gsem[b])

    def wait_gathers(b):
        pltpu.make_async_copy(hap_hbm.at[pl.ds(0, CH)], hbuf2[b],
                              gsem[b]).wait()
        pltpu.make_async_copy(adp_hbm.at[pl.ds(0, CH)], gd2[b],
                              gsem[b]).wait()

    def compute_scatter(cc, b):
        gd, hbuf = gd2[b], hbuf2[b]

        @plsc.parallel_loop(0, CH, 1, unroll=4)
        def erow(e):
            a = hbuf[e, pl.ds(HID, 16)] + gd[e]
            a = jnp.where(a > 0, a, 0.2 * a)
            ev = jnp.exp(a)
            for j in range(HID // 16):
                msg = hbuf[e, pl.ds(16 * j, 16)] * ev[j]
                hbuf[e, pl.ds(16 * j, 16)] = msg
            hbuf[e, pl.ds(HID, 16)] = ev
        pltpu.async_copy(hbuf, ad_sh.at[idxr.at[cc % 4, 1]], ssem[b],
                         add=True)

    def wait_scatters(b):
        pltpu.make_async_copy(zf_hbm.at[pl.ds(0, CH)], hbuf2[b],
                              ssem[b]).wait()

    issue_idx(0, 0)
    issue_idx(1, 1)
    wait_idx(0)
    issue_gathers(0, 0)
    issue_idx(2, 0)
    wait_gathers(0)
    wait_idx(1)
    issue_gathers(1, 1)
    issue_idx(3, 1)
    compute_scatter(0, 0)
    wait_gathers(1)
    wait_scatters(0)
    wait_idx(0)
    issue_gathers(2, 0)
    compute_scatter(1, 1)

    def body(g, carry):
        c0 = 2 * g
        wait_gathers(0)
        wait_scatters(1)
        wait_idx(1)
        issue_gathers(c0 + 1, 1)
        issue_idx(c0 + 2, 0)
        compute_scatter(c0, 0)
        wait_gathers(1)
        wait_scatters(0)
        wait_idx(0)
        issue_gathers(c0 + 2, 0)
        issue_idx(c0 + 3, 1)
        compute_scatter(c0 + 1, 1)
        return carry
    lax.fori_loop(1, (NCHUNK - 1) // 2, body, 0)

    wait_gathers(0)
    wait_scatters(1)
    compute_scatter(NCHUNK - 1, 0)
    wait_idx(1)
    wait_scatters(0)

    plsc.subcore_barrier()
    pltpu.sync_copy(ad_sh.at[pl.ds(r0, RSIZE)],
                    ad_out.at[cid, pl.ds(r0, RSIZE)])


WF = HID + 16


@functools.lru_cache(maxsize=None)
def _make_edge_pass():
    mesh = plsc.VectorSubcoreMesh(
        core_axis_name="c", subcore_axis_name="s",
        num_cores=NC, num_subcores=NS)
    return pl.kernel(
        _edge_pass_body,
        out_type=jax.ShapeDtypeStruct((NC, N, WF), _f32),
        mesh=mesh,
        compiler_params=pltpu.CompilerParams(use_tc_tiling_on_sc=False),
        scratch_types=[
            pltpu.VMEM((4, 2, CH), jnp.int32),
            (pltpu.VMEM((CH, 16), _f32), pltpu.VMEM((CH, 16), _f32)),
            (pltpu.VMEM((CH, WF), _f32), pltpu.VMEM((CH, WF), _f32)),
            pltpu.SemaphoreType.DMA,
            pltpu.SemaphoreType.DMA,
            pltpu.SemaphoreType.DMA,
            pltpu.SemaphoreType.DMA,
            pltpu.SemaphoreType.DMA,
            pltpu.SemaphoreType.DMA,
            pltpu.VMEM_SHARED((N, WF), _f32),
        ],
    )


def kernel(x, edge_index, W1, att_src1, att_dst1, W2, att_src2, att_dst2):
    ei = edge_index.astype(jnp.int32)
    eidx4 = ei.reshape(2, NW, NCHUNK, CH).transpose(1, 2, 0, 3)
    eidx4 = eidx4.reshape(NW * NCHUNK, 2, CH)
    eidx4 = jnp.concatenate(
        [eidx4, jnp.zeros((NPAD, 2, CH), jnp.int32)], axis=0)

    ar = jnp.arange(HID)
    S1 = jnp.zeros((HID, 16), _f32).at[ar, ar // F1].set(att_src1.reshape(-1))
    D1 = jnp.zeros((HID, 16), _f32).at[ar, ar // F1].set(att_dst1.reshape(-1))
    S2 = jnp.broadcast_to(att_src2.reshape(NCLS, 1), (NCLS, 16)).astype(_f32)
    D2 = jnp.broadcast_to(att_dst2.reshape(NCLS, 1), (NCLS, 16)).astype(_f32)
    E16 = jnp.zeros((16, HID), _f32).at[ar // F1, ar].set(1.0)
    E2 = jnp.zeros((16, NCLS), _f32).at[0, :].set(1.0)
    zwf = jnp.zeros((N, WF), _f32)

    R = 1000
    grid = (N // R,)

    hap1, adp1 = pl.pallas_call(
        _proj1_body,
        grid=grid,
        in_specs=[
            pl.BlockSpec((R, IN_DIM), lambda i: (i, 0)),
            pl.BlockSpec((IN_DIM, HID), lambda i: (0, 0)),
            pl.BlockSpec((HID, 16), lambda i: (0, 0)),
            pl.BlockSpec((HID, 16), lambda i: (0, 0)),
        ],
        out_specs=[
            pl.BlockSpec((R, WF), lambda i: (i, 0)),
            pl.BlockSpec((R, 16), lambda i: (i, 0)),
        ],
        out_shape=[
            jax.ShapeDtypeStruct((N, WF), _f32),
            jax.ShapeDtypeStruct((N, 16), _f32),
        ],
    )(x, W1, S1, D1)

    ad1 = _make_edge_pass()(eidx4, hap1, adp1, zwf)

    hap2, adp2 = pl.pallas_call(
        _mid_body,
        grid=grid,
        in_specs=[
            pl.BlockSpec((NC, R, WF), lambda i: (0, i, 0)),
            pl.BlockSpec((HID, NCLS), lambda i: (0, 0)),
            pl.BlockSpec((NCLS, 16), lambda i: (0, 0)),
            pl.BlockSpec((NCLS, 16), lambda i: (0, 0)),
            pl.BlockSpec((16, HID), lambda i: (0, 0)),
        ],
        out_specs=[
            pl.BlockSpec((R, WF), lambda i: (i, 0)),
            pl.BlockSpec((R, 16), lambda i: (i, 0)),
        ],
        out_shape=[
            jax.ShapeDtypeStruct((N, WF), _f32),
            jax.ShapeDtypeStruct((N, 16), _f32),
        ],
    )(ad1, W2, S2, D2, E16)

    ad2 = _make_edge_pass()(eidx4, hap2, adp2, zwf)

    out = pl.pallas_call(
        _final_body,
        grid=grid,
        in_specs=[
            pl.BlockSpec((NC, R, WF), lambda i: (0, i, 0)),
            pl.BlockSpec((16, NCLS), lambda i: (0, 0)),
        ],
        out_specs=pl.BlockSpec((R, NCLS), lambda i: (i, 0)),
        out_shape=jax.ShapeDtypeStruct((N, NCLS), _f32),
    )(ad2, E2)

    return out
